# baseline (speedup 1.0000x reference)
import functools

import jax
import jax.numpy as jnp
from jax.experimental import pallas as pl

N_NODES = 10000
D_FEAT = 128
D_EDGE = 16

EDGE_BLK = 2048
ROW_BLK = 1024
N_PAD = 10240


def _edge_lin_body(ea_ref, we_ref, be_ref, o_ref):
    o_ref[...] = (
        jnp.dot(ea_ref[...], we_ref[...], preferred_element_type=jnp.float32)
        + be_ref[...]
    )


def _edge_lin(edge_attr, We, be):
    E = edge_attr.shape[0]
    grid = (E // EDGE_BLK,)
    return pl.pallas_call(
        _edge_lin_body,
        grid=grid,
        in_specs=[
            pl.BlockSpec((EDGE_BLK, D_EDGE), lambda i: (i, 0)),
            pl.BlockSpec((D_EDGE, D_FEAT), lambda i: (0, 0)),
            pl.BlockSpec((1, D_FEAT), lambda i: (0, 0)),
        ],
        out_specs=pl.BlockSpec((EDGE_BLK, D_FEAT), lambda i: (i, 0)),
        out_shape=jax.ShapeDtypeStruct((E, D_FEAT), jnp.float32),
    )(edge_attr, We, be.reshape(1, D_FEAT))


def _mlp_body(x_ref, aggr_ref, cnt_ref, w1_ref, b1_ref, w2_ref, b2_ref, o_ref):
    aggr = aggr_ref[...] / jnp.maximum(cnt_ref[...], 1.0)
    h = x_ref[...] + aggr
    h = jnp.maximum(
        jnp.dot(h, w1_ref[...], preferred_element_type=jnp.float32) + b1_ref[...],
        0.0,
    )
    o_ref[...] = (
        jnp.dot(h, w2_ref[...], preferred_element_type=jnp.float32) + b2_ref[...]
    )


def _mlp(x, aggr, counts, W1, b1, W2, b2):
    n = x.shape[0]
    grid = (n // ROW_BLK,)
    return pl.pallas_call(
        _mlp_body,
        grid=grid,
        in_specs=[
            pl.BlockSpec((ROW_BLK, D_FEAT), lambda i: (i, 0)),
            pl.BlockSpec((ROW_BLK, D_FEAT), lambda i: (i, 0)),
            pl.BlockSpec((ROW_BLK, 1), lambda i: (i, 0)),
            pl.BlockSpec((D_FEAT, D_FEAT), lambda i: (0, 0)),
            pl.BlockSpec((1, D_FEAT), lambda i: (0, 0)),
            pl.BlockSpec((D_FEAT, D_FEAT), lambda i: (0, 0)),
            pl.BlockSpec((1, D_FEAT), lambda i: (0, 0)),
        ],
        out_specs=pl.BlockSpec((ROW_BLK, D_FEAT), lambda i: (i, 0)),
        out_shape=jax.ShapeDtypeStruct((n, D_FEAT), jnp.float32),
    )(x, aggr, counts, W1, b1.reshape(1, D_FEAT), W2, b2.reshape(1, D_FEAT))


def kernel(x, edge_index, edge_attr, We, be, W1, b1, W2, b2):
    src = edge_index[0]
    dst = edge_index[1]
    e = _edge_lin(edge_attr, We, be)
    msg = jax.nn.relu(x[src] + e)
    summed = jax.ops.segment_sum(msg, dst, num_segments=N_NODES)
    counts = jax.ops.segment_sum(
        jnp.ones((msg.shape[0], 1), msg.dtype), dst, num_segments=N_NODES
    )
    xp = jnp.pad(x, ((0, N_PAD - N_NODES), (0, 0)))
    sp = jnp.pad(summed, ((0, N_PAD - N_NODES), (0, 0)))
    cp = jnp.pad(counts, ((0, N_PAD - N_NODES), (0, 0)))
    out = _mlp(xp, sp, cp, W1, b1, W2, b2)
    return out[:N_NODES]


# speedup vs baseline: 3.0441x; 3.0441x over previous
import functools

import jax
import jax.numpy as jnp
from jax import lax
from jax.experimental import pallas as pl
from jax.experimental.pallas import tpu as pltpu
from jax.experimental.pallas import tpu_sc as plsc

N_NODES = 10000
N_EDGES = 320000
D_FEAT = 128
D_EDGE = 16

EDGE_BLK = 2048

NC = 2
NS = 16
NW = NC * NS
C = 64
NCHUNK = N_EDGES // C
CHUNK_ITERS = (NCHUNK + NW - 1) // NW

N_PAD = 10240
TILE_ROWS = N_PAD // NS
ROW_BLK = N_PAD // 8


def _edge_lin_body(ea_ref, we_ref, be_ref, o_ref):
    o_ref[...] = (
        jnp.dot(ea_ref[...], we_ref[...], preferred_element_type=jnp.float32)
        + be_ref[...]
    )


def _edge_lin(edge_attr, We, be):
    E = edge_attr.shape[0]
    return pl.pallas_call(
        _edge_lin_body,
        grid=(E // EDGE_BLK,),
        in_specs=[
            pl.BlockSpec((EDGE_BLK, D_EDGE), lambda i: (i, 0)),
            pl.BlockSpec((D_EDGE, D_FEAT), lambda i: (0, 0)),
            pl.BlockSpec((1, D_FEAT), lambda i: (0, 0)),
        ],
        out_specs=pl.BlockSpec((EDGE_BLK, D_FEAT), lambda i: (i, 0)),
        out_shape=jax.ShapeDtypeStruct((E, D_FEAT), jnp.float32),
    )(edge_attr, We, be.reshape(1, D_FEAT))


def _sc_body(x_hbm, src_hbm, dst_hbm, e_hbm, acc_out, cnt_out,
             idx2, xrows, erows, hist, acc_sh, sem):
    c = lax.axis_index("c")
    s = lax.axis_index("s")
    wid = s * NC + c

    zeros16 = jnp.zeros((16,), jnp.float32)

    def zero_hist(k, _):
        hist[pl.ds(k * 16, 16)] = zeros16
        return 0

    lax.fori_loop(0, (N_PAD + 16) // 16, zero_hist, 0)

    def zero_row(r, _):
        for j in range(D_FEAT // 16):
            erows[r, pl.ds(j * 16, 16)] = zeros16
        return 0

    lax.fori_loop(0, C, zero_row, 0)

    def zero_chunk(k, _):
        pltpu.sync_copy(erows, acc_sh.at[pl.ds(s * TILE_ROWS + k * C, C)])
        return 0

    lax.fori_loop(0, TILE_ROWS // C, zero_chunk, 0)
    plsc.subcore_barrier()

    def chunk_body(i, _):
        ck = i * NW + wid

        @pl.when(ck < NCHUNK)
        def _():
            base = ck * C
            pltpu.sync_copy(src_hbm.at[pl.ds(base, C)], idx2.at[0])
            pltpu.sync_copy(dst_hbm.at[pl.ds(base, C)], idx2.at[1])
            gat = pltpu.async_copy(x_hbm.at[idx2.at[0]], xrows, sem)
            pltpu.sync_copy(e_hbm.at[pl.ds(base, C)], erows)

            e0 = jnp.where(lax.iota(jnp.int32, 16) == 0, 1.0, 0.0)
            for k in range(C // 16):
                dv = idx2[1, pl.ds(k * 16, 16)]
                for l in range(16):
                    d = dv[l]
                    hist[pl.ds(d, 16)] = hist[pl.ds(d, 16)] + e0

            gat.wait()

            def row_body(r, _):
                for j in range(D_FEAT // 16):
                    sl = pl.ds(j * 16, 16)
                    xrows[r, sl] = jnp.maximum(xrows[r, sl] + erows[r, sl], 0.0)
                return 0

            lax.fori_loop(0, C, row_body, 0)
            pltpu.sync_copy(xrows, acc_sh.at[idx2.at[1]], add=True)

        return 0

    lax.fori_loop(0, CHUNK_ITERS, chunk_body, 0)
    plsc.subcore_barrier()

    def read_chunk(k, _):
        row0 = s * TILE_ROWS + k * C
        pltpu.sync_copy(acc_sh.at[pl.ds(row0, C)], erows)
        pltpu.sync_copy(erows, acc_out.at[pl.ds(c * N_PAD + row0, C)])
        return 0

    lax.fori_loop(0, TILE_ROWS // C, read_chunk, 0)
    pltpu.sync_copy(hist.at[pl.ds(0, N_PAD)], cnt_out.at[wid])


def _sc_aggregate(x, src, dst, e):
    mesh = plsc.VectorSubcoreMesh(
        core_axis_name="c", subcore_axis_name="s", num_cores=NC, num_subcores=NS
    )
    f = pl.kernel(
        _sc_body,
        out_type=(
            jax.ShapeDtypeStruct((NC * N_PAD, D_FEAT), jnp.float32),
            jax.ShapeDtypeStruct((NW, N_PAD), jnp.float32),
        ),
        mesh=mesh,
        scratch_types=[
            pltpu.VMEM((2, C), jnp.int32),
            pltpu.VMEM((C, D_FEAT), jnp.float32),
            pltpu.VMEM((C, D_FEAT), jnp.float32),
            pltpu.VMEM((N_PAD + 16,), jnp.float32),
            pltpu.VMEM_SHARED((N_PAD, D_FEAT), jnp.float32),
            pltpu.SemaphoreType.DMA,
        ],
    )
    return f(x, src, dst, e)


def _mlp_body(x_ref, a0_ref, a1_ref, cnt_ref,
              w1_ref, b1_ref, w2_ref, b2_ref, o_ref):
    cnt = jnp.maximum(cnt_ref[...], 1.0)
    aggr = (a0_ref[...] + a1_ref[...]) / cnt
    h = x_ref[...] + aggr
    h = jnp.maximum(
        jnp.dot(h, w1_ref[...], preferred_element_type=jnp.float32) + b1_ref[...],
        0.0,
    )
    o_ref[...] = (
        jnp.dot(h, w2_ref[...], preferred_element_type=jnp.float32) + b2_ref[...]
    )


def _mlp(xp, acc, cnt, W1, b1, W2, b2):
    nb = N_PAD // ROW_BLK
    return pl.pallas_call(
        _mlp_body,
        grid=(nb,),
        in_specs=[
            pl.BlockSpec((ROW_BLK, D_FEAT), lambda i: (i, 0)),
            pl.BlockSpec((ROW_BLK, D_FEAT), lambda i: (i, 0)),
            pl.BlockSpec((ROW_BLK, D_FEAT), lambda i, nb=nb: (i + nb, 0)),
            pl.BlockSpec((ROW_BLK, 1), lambda i: (i, 0)),
            pl.BlockSpec((D_FEAT, D_FEAT), lambda i: (0, 0)),
            pl.BlockSpec((1, D_FEAT), lambda i: (0, 0)),
            pl.BlockSpec((D_FEAT, D_FEAT), lambda i: (0, 0)),
            pl.BlockSpec((1, D_FEAT), lambda i: (0, 0)),
        ],
        out_specs=pl.BlockSpec((ROW_BLK, D_FEAT), lambda i: (i, 0)),
        out_shape=jax.ShapeDtypeStruct((N_PAD, D_FEAT), jnp.float32),
    )(xp, acc, acc, cnt, W1, b1.reshape(1, D_FEAT), W2, b2.reshape(1, D_FEAT))


def kernel(x, edge_index, edge_attr, We, be, W1, b1, W2, b2):
    src = edge_index[0].astype(jnp.int32)
    dst = edge_index[1].astype(jnp.int32)
    e = _edge_lin(edge_attr, We, be)
    acc, cnt = _sc_aggregate(x, src, dst, e)
    counts = cnt.sum(axis=0).reshape(N_PAD, 1)
    xp = jnp.pad(x, ((0, N_PAD - N_NODES), (0, 0)))
    out = _mlp(xp, acc, counts, W1, b1, W2, b2)
    return out[:N_NODES]


# speedup vs baseline: 3.3196x; 1.0905x over previous
import functools

import jax
import jax.numpy as jnp
from jax import lax
from jax.experimental import pallas as pl
from jax.experimental.pallas import tpu as pltpu
from jax.experimental.pallas import tpu_sc as plsc

N_NODES = 10000
N_EDGES = 320000
D_FEAT = 128
D_EDGE = 16

EDGE_BLK = 2048

NC = 2
NS = 16
NW = NC * NS
C = 96
E_PAD_SC = ((N_EDGES + C - 1) // C) * C
NCHUNK = E_PAD_SC // C
CHUNK_ITERS = (NCHUNK + NW - 1) // NW
E_PAD_TC = ((E_PAD_SC + EDGE_BLK - 1) // EDGE_BLK) * EDGE_BLK
ZB = 64

N_PAD = 10240
TILE_ROWS = N_PAD // NS
ROW_BLK = N_PAD // 8


def _edge_lin_body(ea_ref, we_ref, be_ref, o_ref):
    o_ref[...] = (
        jnp.dot(ea_ref[...], we_ref[...], preferred_element_type=jnp.float32)
        + be_ref[...]
    )


def _edge_lin(edge_attr, We, be):
    E = edge_attr.shape[0]
    return pl.pallas_call(
        _edge_lin_body,
        grid=(E // EDGE_BLK,),
        in_specs=[
            pl.BlockSpec((EDGE_BLK, D_EDGE), lambda i: (i, 0)),
            pl.BlockSpec((D_EDGE, D_FEAT), lambda i: (0, 0)),
            pl.BlockSpec((1, D_FEAT), lambda i: (0, 0)),
        ],
        out_specs=pl.BlockSpec((EDGE_BLK, D_FEAT), lambda i: (i, 0)),
        out_shape=jax.ShapeDtypeStruct((E, D_FEAT), jnp.float32),
    )(edge_attr, We, be.reshape(1, D_FEAT))


def _sc_body(x_hbm, ei3_hbm, e_hbm, acc_out, cnt_out,
             idx2, xrows, erows, hist, acc_sh, sem):
    c = lax.axis_index("c")
    s = lax.axis_index("s")
    wid = s * NC + c

    zeros16 = jnp.zeros((16,), jnp.float32)

    def zero_hist(k, _):
        hist[pl.ds(k * 16, 16)] = zeros16
        return 0

    lax.fori_loop(0, (N_PAD + 16) // 16, zero_hist, 0)

    def zero_row(r, _):
        for j in range(D_FEAT // 16):
            erows[r, pl.ds(j * 16, 16)] = zeros16
        return 0

    lax.fori_loop(0, C, zero_row, 0)

    def zero_chunk(k, _):
        pltpu.sync_copy(erows.at[pl.ds(0, ZB)],
                        acc_sh.at[pl.ds(s * TILE_ROWS + k * ZB, ZB)])
        return 0

    lax.fori_loop(0, TILE_ROWS // ZB, zero_chunk, 0)
    plsc.subcore_barrier()

    def chunk_body(i, _):
        ck = i * NW + wid

        @pl.when(ck < NCHUNK)
        def _():
            base = ck * C
            pltpu.sync_copy(ei3_hbm.at[ck], idx2)
            gat = pltpu.async_copy(x_hbm.at[idx2.at[0]], xrows, sem)
            pltpu.sync_copy(e_hbm.at[pl.ds(base, C)], erows)

            e0 = jnp.where(lax.iota(jnp.int32, 16) == 0, 1.0, 0.0)
            for k in range(C // 16):
                dv = idx2[1, pl.ds(k * 16, 16)]
                for l in range(16):
                    d = dv[l]
                    hist[pl.ds(d, 16)] = hist[pl.ds(d, 16)] + e0

            gat.wait()

            def row_body(r, _):
                for j in range(D_FEAT // 16):
                    sl = pl.ds(j * 16, 16)
                    xrows[r, sl] = jnp.maximum(xrows[r, sl] + erows[r, sl], 0.0)
                return 0

            lax.fori_loop(0, C, row_body, 0)
            pltpu.sync_copy(xrows, acc_sh.at[idx2.at[1]], add=True)

        return 0

    lax.fori_loop(0, CHUNK_ITERS, chunk_body, 0)
    plsc.subcore_barrier()

    def read_chunk(k, _):
        row0 = s * TILE_ROWS + k * ZB
        pltpu.sync_copy(acc_sh.at[pl.ds(row0, ZB)], erows.at[pl.ds(0, ZB)])
        pltpu.sync_copy(erows.at[pl.ds(0, ZB)],
                        acc_out.at[pl.ds(c * N_PAD + row0, ZB)])
        return 0

    lax.fori_loop(0, TILE_ROWS // ZB, read_chunk, 0)
    pltpu.sync_copy(hist.at[pl.ds(0, N_PAD)], cnt_out.at[wid])


def _sc_aggregate(x, ei3, e):
    mesh = plsc.VectorSubcoreMesh(
        core_axis_name="c", subcore_axis_name="s", num_cores=NC, num_subcores=NS
    )
    f = pl.kernel(
        _sc_body,
        out_type=(
            jax.ShapeDtypeStruct((NC * N_PAD, D_FEAT), jnp.float32),
            jax.ShapeDtypeStruct((NW, N_PAD), jnp.float32),
        ),
        mesh=mesh,
        scratch_types=[
            pltpu.VMEM((2, C), jnp.int32),
            pltpu.VMEM((C, D_FEAT), jnp.float32),
            pltpu.VMEM((C, D_FEAT), jnp.float32),
            pltpu.VMEM((N_PAD + 16,), jnp.float32),
            pltpu.VMEM_SHARED((N_PAD, D_FEAT), jnp.float32),
            pltpu.SemaphoreType.DMA,
        ],
    )
    return f(x, ei3, e)


def _mlp_body(x_ref, a0_ref, a1_ref, cnt_ref,
              w1_ref, b1_ref, w2_ref, b2_ref, o_ref):
    cnt = jnp.maximum(cnt_ref[...], 1.0)
    aggr = (a0_ref[...] + a1_ref[...]) / cnt
    h = x_ref[...] + aggr
    h = jnp.maximum(
        jnp.dot(h, w1_ref[...], preferred_element_type=jnp.float32) + b1_ref[...],
        0.0,
    )
    o_ref[...] = (
        jnp.dot(h, w2_ref[...], preferred_element_type=jnp.float32) + b2_ref[...]
    )


def _mlp(xp, acc, cnt, W1, b1, W2, b2):
    nb = N_PAD // ROW_BLK
    return pl.pallas_call(
        _mlp_body,
        grid=(nb,),
        in_specs=[
            pl.BlockSpec((ROW_BLK, D_FEAT), lambda i: (i, 0)),
            pl.BlockSpec((ROW_BLK, D_FEAT), lambda i: (i, 0)),
            pl.BlockSpec((ROW_BLK, D_FEAT), lambda i, nb=nb: (i + nb, 0)),
            pl.BlockSpec((ROW_BLK, 1), lambda i: (i, 0)),
            pl.BlockSpec((D_FEAT, D_FEAT), lambda i: (0, 0)),
            pl.BlockSpec((1, D_FEAT), lambda i: (0, 0)),
            pl.BlockSpec((D_FEAT, D_FEAT), lambda i: (0, 0)),
            pl.BlockSpec((1, D_FEAT), lambda i: (0, 0)),
        ],
        out_specs=pl.BlockSpec((ROW_BLK, D_FEAT), lambda i: (i, 0)),
        out_shape=jax.ShapeDtypeStruct((N_PAD, D_FEAT), jnp.float32),
    )(xp, acc, acc, cnt, W1, b1.reshape(1, D_FEAT), W2, b2.reshape(1, D_FEAT))


def kernel(x, edge_index, edge_attr, We, be, W1, b1, W2, b2):
    ei = edge_index.astype(jnp.int32)
    pad = E_PAD_SC - N_EDGES
    srcp = jnp.concatenate([ei[0], jnp.zeros((pad,), jnp.int32)])
    dstp = jnp.concatenate([ei[1], jnp.full((pad,), N_NODES, jnp.int32)])
    ei3 = jnp.stack([srcp, dstp]).reshape(2, NCHUNK, C).transpose(1, 0, 2)
    eap = jnp.pad(edge_attr, ((0, E_PAD_TC - N_EDGES), (0, 0)))
    e = _edge_lin(eap, We, be)
    acc, cnt = _sc_aggregate(x, ei3, e)
    counts = cnt.sum(axis=0).reshape(N_PAD, 1)
    xp = jnp.pad(x, ((0, N_PAD - N_NODES), (0, 0)))
    out = _mlp(xp, acc, counts, W1, b1, W2, b2)
    return out[:N_NODES]


# speedup vs baseline: 3.4272x; 1.0324x over previous
import functools

import jax
import jax.numpy as jnp
from jax import lax
from jax.experimental import pallas as pl
from jax.experimental.pallas import tpu as pltpu
from jax.experimental.pallas import tpu_sc as plsc

N_NODES = 10000
N_EDGES = 320000
D_FEAT = 128
D_EDGE = 16

EDGE_BLK = 2048

NC = 2
NS = 16
NW = NC * NS
C = 96
E_PAD_SC = ((N_EDGES + C - 1) // C) * C
NCHUNK = E_PAD_SC // C
CHUNK_ITERS = (NCHUNK + NW - 1) // NW
E_PAD_TC = ((E_PAD_SC + EDGE_BLK - 1) // EDGE_BLK) * EDGE_BLK

N_PAD = 10240
ROW_BLK = N_PAD // 8
N_ACC = 10112
TILE_ROWS = N_ACC // NS
ZB = 64
ZT = TILE_ROWS - 9 * ZB


def _edge_lin_body(ea_ref, we_ref, be_ref, o_ref):
    o_ref[...] = (
        jnp.dot(ea_ref[...], we_ref[...], preferred_element_type=jnp.float32)
        + be_ref[...]
    )


def _edge_lin(edge_attr, We, be):
    E = edge_attr.shape[0]
    return pl.pallas_call(
        _edge_lin_body,
        grid=(E // EDGE_BLK,),
        in_specs=[
            pl.BlockSpec((EDGE_BLK, D_EDGE), lambda i: (i, 0)),
            pl.BlockSpec((D_EDGE, D_FEAT), lambda i: (0, 0)),
            pl.BlockSpec((1, D_FEAT), lambda i: (0, 0)),
        ],
        out_specs=pl.BlockSpec((EDGE_BLK, D_FEAT), lambda i: (i, 0)),
        out_shape=jax.ShapeDtypeStruct((E, D_FEAT), jnp.float32),
    )(edge_attr, We, be.reshape(1, D_FEAT))


def _sc_body(x_hbm, ei3_hbm, e_hbm, acc_out, cnt_out,
             idx2, dstc, xrows, erows, hist, acc_sh, sem, sem2):
    c = lax.axis_index("c")
    s = lax.axis_index("s")
    wid = s * NC + c

    zeros16 = jnp.zeros((16,), jnp.float32)

    def zero_hist(k, _):
        hist[pl.ds(k * 16, 16)] = zeros16
        return 0

    lax.fori_loop(0, (N_PAD + 16) // 16, zero_hist, 0)

    def zero_row(r, _):
        for j in range(D_FEAT // 16):
            erows[r, pl.ds(j * 16, 16)] = zeros16
        return 0

    lax.fori_loop(0, C, zero_row, 0)

    def zero_chunk(k, _):
        pltpu.sync_copy(erows.at[pl.ds(0, ZB)],
                        acc_sh.at[pl.ds(s * TILE_ROWS + k * ZB, ZB)])
        return 0

    lax.fori_loop(0, 9, zero_chunk, 0)
    pltpu.sync_copy(erows.at[pl.ds(0, ZT)],
                    acc_sh.at[pl.ds(s * TILE_ROWS + 9 * ZB, ZT)])
    plsc.subcore_barrier()

    def chunk_body(i, _):
        ck = i * NW + wid

        @pl.when(ck < NCHUNK)
        def _():
            base = ck * C
            pltpu.sync_copy(ei3_hbm.at[ck], idx2)
            pltpu.sync_copy(e_hbm.at[pl.ds(base, C)], erows)

            @pl.when(i > 0)
            def _():
                pltpu.make_async_copy(e_hbm.at[pl.ds(0, C)], xrows, sem2).wait()

            gat = pltpu.async_copy(x_hbm.at[idx2.at[0]], xrows, sem)

            e0 = jnp.where(lax.iota(jnp.int32, 16) == 0, 1.0, 0.0)
            for k in range(C // 16):
                sl = pl.ds(k * 16, 16)
                dv = idx2[1, sl]
                dstc[1, sl] = dv
                for l in range(16):
                    d = dv[l]
                    hist[pl.ds(d, 16)] = hist[pl.ds(d, 16)] + e0

            gat.wait()

            def row_body(r, _):
                for j in range(D_FEAT // 16):
                    sl = pl.ds(j * 16, 16)
                    xrows[r, sl] = jnp.maximum(xrows[r, sl] + erows[r, sl], 0.0)
                return 0

            lax.fori_loop(0, C, row_body, 0)
            pltpu.async_copy(xrows, acc_sh.at[dstc.at[1]], sem2, add=True)

        return 0

    lax.fori_loop(0, CHUNK_ITERS, chunk_body, 0)
    pltpu.make_async_copy(e_hbm.at[pl.ds(0, C)], xrows, sem2).wait()
    plsc.subcore_barrier()

    def read_chunk(k, _):
        row0 = s * TILE_ROWS + k * ZB
        pltpu.sync_copy(acc_sh.at[pl.ds(row0, ZB)], erows.at[pl.ds(0, ZB)])
        pltpu.sync_copy(erows.at[pl.ds(0, ZB)],
                        acc_out.at[pl.ds(c * N_PAD + row0, ZB)])
        return 0

    lax.fori_loop(0, 9, read_chunk, 0)
    row9 = s * TILE_ROWS + 9 * ZB
    pltpu.sync_copy(acc_sh.at[pl.ds(row9, ZT)], erows.at[pl.ds(0, ZT)])
    pltpu.sync_copy(erows.at[pl.ds(0, ZT)],
                    acc_out.at[pl.ds(c * N_PAD + row9, ZT)])
    pltpu.sync_copy(hist.at[pl.ds(0, N_PAD)], cnt_out.at[wid])


def _sc_aggregate(x, ei3, e):
    mesh = plsc.VectorSubcoreMesh(
        core_axis_name="c", subcore_axis_name="s", num_cores=NC, num_subcores=NS
    )
    f = pl.kernel(
        _sc_body,
        out_type=(
            jax.ShapeDtypeStruct((NC * N_PAD, D_FEAT), jnp.float32),
            jax.ShapeDtypeStruct((NW, N_PAD), jnp.float32),
        ),
        mesh=mesh,
        scratch_types=[
            pltpu.VMEM((2, C), jnp.int32),
            pltpu.VMEM((2, C), jnp.int32),
            pltpu.VMEM((C, D_FEAT), jnp.float32),
            pltpu.VMEM((C, D_FEAT), jnp.float32),
            pltpu.VMEM((N_PAD + 16,), jnp.float32),
            pltpu.VMEM_SHARED((N_ACC, D_FEAT), jnp.float32),
            pltpu.SemaphoreType.DMA,
            pltpu.SemaphoreType.DMA,
        ],
    )
    return f(x, ei3, e)


def _mlp_body(x_ref, a0_ref, a1_ref, cnt_ref,
              w1_ref, b1_ref, w2_ref, b2_ref, o_ref):
    cnt = jnp.maximum(cnt_ref[...], 1.0)
    aggr = (a0_ref[...] + a1_ref[...]) / cnt
    h = x_ref[...] + aggr
    h = jnp.maximum(
        jnp.dot(h, w1_ref[...], preferred_element_type=jnp.float32) + b1_ref[...],
        0.0,
    )
    o_ref[...] = (
        jnp.dot(h, w2_ref[...], preferred_element_type=jnp.float32) + b2_ref[...]
    )


def _mlp(xp, acc, cnt, W1, b1, W2, b2):
    nb = N_PAD // ROW_BLK
    return pl.pallas_call(
        _mlp_body,
        grid=(nb,),
        in_specs=[
            pl.BlockSpec((ROW_BLK, D_FEAT), lambda i: (i, 0)),
            pl.BlockSpec((ROW_BLK, D_FEAT), lambda i: (i, 0)),
            pl.BlockSpec((ROW_BLK, D_FEAT), lambda i, nb=nb: (i + nb, 0)),
            pl.BlockSpec((ROW_BLK, 1), lambda i: (i, 0)),
            pl.BlockSpec((D_FEAT, D_FEAT), lambda i: (0, 0)),
            pl.BlockSpec((1, D_FEAT), lambda i: (0, 0)),
            pl.BlockSpec((D_FEAT, D_FEAT), lambda i: (0, 0)),
            pl.BlockSpec((1, D_FEAT), lambda i: (0, 0)),
        ],
        out_specs=pl.BlockSpec((ROW_BLK, D_FEAT), lambda i: (i, 0)),
        out_shape=jax.ShapeDtypeStruct((N_PAD, D_FEAT), jnp.float32),
    )(xp, acc, acc, cnt, W1, b1.reshape(1, D_FEAT), W2, b2.reshape(1, D_FEAT))


def kernel(x, edge_index, edge_attr, We, be, W1, b1, W2, b2):
    ei = edge_index.astype(jnp.int32)
    pad = E_PAD_SC - N_EDGES
    srcp = jnp.concatenate([ei[0], jnp.zeros((pad,), jnp.int32)])
    dstp = jnp.concatenate([ei[1], jnp.full((pad,), N_NODES, jnp.int32)])
    ei3 = jnp.stack([srcp, dstp]).reshape(2, NCHUNK, C).transpose(1, 0, 2)
    eap = jnp.pad(edge_attr, ((0, E_PAD_TC - N_EDGES), (0, 0)))
    e = _edge_lin(eap, We, be)
    acc, cnt = _sc_aggregate(x, ei3, e)
    counts = cnt.sum(axis=0).reshape(N_PAD, 1)
    xp = jnp.pad(x, ((0, N_PAD - N_NODES), (0, 0)))
    out = _mlp(xp, acc, counts, W1, b1, W2, b2)
    return out[:N_NODES]


# speedup vs baseline: 4.0163x; 1.1719x over previous
import functools

import jax
import jax.numpy as jnp
from jax import lax
from jax.experimental import pallas as pl
from jax.experimental.pallas import tpu as pltpu
from jax.experimental.pallas import tpu_sc as plsc

N_NODES = 10000
N_EDGES = 320000
D_FEAT = 128
D_EDGE = 16

EDGE_BLK = 2048

NC = 2
NS = 16
NW = NC * NS
C = 96
E_PAD_SC = ((N_EDGES + C - 1) // C) * C
NCHUNK = E_PAD_SC // C
CHUNK_ITERS = (NCHUNK + NW - 1) // NW
E_PAD_TC = ((E_PAD_SC + EDGE_BLK - 1) // EDGE_BLK) * EDGE_BLK

N_PAD = 10240
ROW_BLK = N_PAD // 8
N_ACC = 10112
TILE_ROWS = N_ACC // NS
ZB = 64
ZT = TILE_ROWS - 9 * ZB


_EPERM = []
for _m in range(D_FEAT // 32):
    for _i in range(16):
        _EPERM.append(32 * _m + _i)
        _EPERM.append(32 * _m + 16 + _i)
_EPERM = tuple(_EPERM)


def _edge_lin_body(ea_ref, we_ref, be_ref, o_ref):
    o_ref[...] = (
        jnp.dot(ea_ref[...], we_ref[...], preferred_element_type=jnp.float32)
        + be_ref[...]
    )


def _edge_lin(edge_attr, We, be):
    E = edge_attr.shape[0]
    return pl.pallas_call(
        _edge_lin_body,
        grid=(E // EDGE_BLK,),
        in_specs=[
            pl.BlockSpec((EDGE_BLK, D_EDGE), lambda i: (i, 0)),
            pl.BlockSpec((D_EDGE, D_FEAT), lambda i: (0, 0)),
            pl.BlockSpec((1, D_FEAT), lambda i: (0, 0)),
        ],
        out_specs=pl.BlockSpec((EDGE_BLK, D_FEAT), lambda i: (i, 0)),
        out_shape=jax.ShapeDtypeStruct((E, D_FEAT), jnp.float32),
    )(edge_attr, We, be.reshape(1, D_FEAT))


def _sc_body(x_hbm, ei3_hbm, e_hbm, acc_out, cnt_out,
             idx2, dstc, xrows, erows, hist, acc_sh, sem, sem2, sem3, sem4):
    c = lax.axis_index("c")
    s = lax.axis_index("s")
    wid = s * NC + c

    zeros16 = jnp.zeros((16,), jnp.float32)

    def zero_hist(k, _):
        hist[pl.ds(k * 16, 16)] = zeros16
        return 0

    lax.fori_loop(0, (N_PAD + 16) // 16, zero_hist, 0)

    def zero_row(r, _):
        for j in range(D_FEAT // 16):
            xrows[r, pl.ds(j * 16, 16)] = zeros16
        return 0

    lax.fori_loop(0, C, zero_row, 0)

    def zero_chunk(k, _):
        pltpu.sync_copy(xrows.at[pl.ds(0, ZB)],
                        acc_sh.at[pl.ds(s * TILE_ROWS + k * ZB, ZB)])
        return 0

    lax.fori_loop(0, 9, zero_chunk, 0)
    pltpu.sync_copy(xrows.at[pl.ds(0, ZT)],
                    acc_sh.at[pl.ds(s * TILE_ROWS + 9 * ZB, ZT)])
    plsc.subcore_barrier()

    def chunk_body(i, _):
        ck = i * NW + wid

        @pl.when(ck < NCHUNK)
        def _():
            base = ck * C
            ia = pltpu.async_copy(ei3_hbm.at[ck], idx2, sem3)
            eb = pltpu.async_copy(e_hbm.at[pl.ds(base, C)], erows, sem4)

            @pl.when(i > 0)
            def _():
                pltpu.make_async_copy(x_hbm.at[pl.ds(0, C)], xrows, sem2).wait()

            ia.wait()
            gat = pltpu.async_copy(x_hbm.at[idx2.at[0]], xrows, sem)

            e0 = jnp.where(lax.iota(jnp.int32, 16) == 0, 1.0, 0.0)
            for k in range(C // 16):
                sl = pl.ds(k * 16, 16)
                dv = idx2[1, sl]
                dstc[1, sl] = dv
                for l in range(16):
                    d = dv[l]
                    hist[pl.ds(d, 16)] = hist[pl.ds(d, 16)] + e0

            eb.wait()
            gat.wait()

            def row_body(r, _):
                for j in range(D_FEAT // 16):
                    sl = pl.ds(j * 16, 16)
                    xrows[r, sl] = jnp.maximum(xrows[r, sl] + erows[r, sl], 0.0)
                return 0

            lax.fori_loop(0, C, row_body, 0)
            pltpu.async_copy(xrows, acc_sh.at[dstc.at[1]], sem2, add=True)

        return 0

    lax.fori_loop(0, CHUNK_ITERS, chunk_body, 0)
    pltpu.make_async_copy(x_hbm.at[pl.ds(0, C)], xrows, sem2).wait()
    plsc.subcore_barrier()

    def read_chunk(k, _):
        row0 = s * TILE_ROWS + k * ZB
        pltpu.sync_copy(acc_sh.at[pl.ds(row0, ZB)], xrows.at[pl.ds(0, ZB)])
        pltpu.sync_copy(xrows.at[pl.ds(0, ZB)],
                        acc_out.at[pl.ds(c * N_PAD + row0, ZB)])
        return 0

    lax.fori_loop(0, 9, read_chunk, 0)
    row9 = s * TILE_ROWS + 9 * ZB
    pltpu.sync_copy(acc_sh.at[pl.ds(row9, ZT)], xrows.at[pl.ds(0, ZT)])
    pltpu.sync_copy(xrows.at[pl.ds(0, ZT)],
                    acc_out.at[pl.ds(c * N_PAD + row9, ZT)])
    pltpu.sync_copy(hist.at[pl.ds(0, N_PAD)], cnt_out.at[wid])


def _sc_aggregate(x, ei3, e):
    mesh = plsc.VectorSubcoreMesh(
        core_axis_name="c", subcore_axis_name="s", num_cores=NC, num_subcores=NS
    )
    f = pl.kernel(
        _sc_body,
        out_type=(
            jax.ShapeDtypeStruct((NC * N_PAD, D_FEAT), jnp.float32),
            jax.ShapeDtypeStruct((NW, N_PAD), jnp.float32),
        ),
        mesh=mesh,
        scratch_types=[
            pltpu.VMEM((2, C), jnp.int32),
            pltpu.VMEM((2, C), jnp.int32),
            pltpu.VMEM((C, D_FEAT), jnp.float32),
            pltpu.VMEM((C, D_FEAT), jnp.float32),
            pltpu.VMEM((N_PAD + 16,), jnp.float32),
            pltpu.VMEM_SHARED((N_ACC, D_FEAT), jnp.float32),
            pltpu.SemaphoreType.DMA,
            pltpu.SemaphoreType.DMA,
            pltpu.SemaphoreType.DMA,
            pltpu.SemaphoreType.DMA,
        ],
    )
    return f(x, ei3, e)


def _mlp_body(x_ref, a0_ref, a1_ref, cnt_ref,
              w1_ref, b1_ref, w2_ref, b2_ref, o_ref):
    cnt = jnp.maximum(cnt_ref[...], 1.0)
    aggr = (a0_ref[...] + a1_ref[...]) / cnt
    h = x_ref[...] + aggr
    h = jnp.maximum(
        jnp.dot(h, w1_ref[...], preferred_element_type=jnp.float32) + b1_ref[...],
        0.0,
    )
    o_ref[...] = (
        jnp.dot(h, w2_ref[...], preferred_element_type=jnp.float32) + b2_ref[...]
    )


def _mlp(xp, acc, cnt, W1, b1, W2, b2):
    nb = N_PAD // ROW_BLK
    return pl.pallas_call(
        _mlp_body,
        grid=(nb,),
        in_specs=[
            pl.BlockSpec((ROW_BLK, D_FEAT), lambda i: (i, 0)),
            pl.BlockSpec((ROW_BLK, D_FEAT), lambda i: (i, 0)),
            pl.BlockSpec((ROW_BLK, D_FEAT), lambda i, nb=nb: (i + nb, 0)),
            pl.BlockSpec((ROW_BLK, 1), lambda i: (i, 0)),
            pl.BlockSpec((D_FEAT, D_FEAT), lambda i: (0, 0)),
            pl.BlockSpec((1, D_FEAT), lambda i: (0, 0)),
            pl.BlockSpec((D_FEAT, D_FEAT), lambda i: (0, 0)),
            pl.BlockSpec((1, D_FEAT), lambda i: (0, 0)),
        ],
        out_specs=pl.BlockSpec((ROW_BLK, D_FEAT), lambda i: (i, 0)),
        out_shape=jax.ShapeDtypeStruct((N_PAD, D_FEAT), jnp.float32),
    )(xp, acc, acc, cnt, W1, b1.reshape(1, D_FEAT), W2, b2.reshape(1, D_FEAT))


def kernel(x, edge_index, edge_attr, We, be, W1, b1, W2, b2):
    ei = edge_index.astype(jnp.int32)
    pad = E_PAD_SC - N_EDGES
    srcp = jnp.concatenate([ei[0], jnp.zeros((pad,), jnp.int32)])
    dstp = jnp.concatenate([ei[1], jnp.full((pad,), N_NODES, jnp.int32)])
    ei3 = jnp.stack([srcp, dstp]).reshape(2, NCHUNK, C).transpose(1, 0, 2)
    eap = jnp.pad(edge_attr, ((0, E_PAD_TC - N_EDGES), (0, 0)))
    e = _edge_lin(eap, We, be)
    acc, cnt = _sc_aggregate(x, ei3, e)
    counts = cnt.sum(axis=0).reshape(N_PAD, 1)
    xp = jnp.pad(x, ((0, N_PAD - N_NODES), (0, 0)))
    out = _mlp(xp, acc, counts, W1, b1, W2, b2)
    return out[:N_NODES]


# speedup vs baseline: 4.4508x; 1.1082x over previous
import functools

import jax
import jax.numpy as jnp
from jax import lax
from jax.experimental import pallas as pl
from jax.experimental.pallas import tpu as pltpu
from jax.experimental.pallas import tpu_sc as plsc

N_NODES = 10000
N_EDGES = 320000
D_FEAT = 128
D_EDGE = 16

EDGE_BLK = 8192

NC = 2
NS = 16
NW = NC * NS
C = 96
E_PAD_SC = ((N_EDGES + C - 1) // C) * C
NCHUNK = E_PAD_SC // C
CHUNK_ITERS = (NCHUNK + NW - 1) // NW
E_PAD_TC = ((E_PAD_SC + EDGE_BLK - 1) // EDGE_BLK) * EDGE_BLK

N_PAD = 10240
ROW_BLK = N_PAD // 8
N_ACC = 10112
TILE_ROWS = N_ACC // NS
ZB = 64
ZT = TILE_ROWS - 9 * ZB


_EPERM = []
for _m in range(D_FEAT // 32):
    for _i in range(16):
        _EPERM.append(32 * _m + _i)
        _EPERM.append(32 * _m + 16 + _i)
_EPERM = tuple(_EPERM)


def _edge_lin_body(ea_ref, we_ref, be_ref, o_ref):
    o_ref[...] = (
        jnp.dot(ea_ref[...], we_ref[...], preferred_element_type=jnp.float32)
        + be_ref[...]
    )


def _edge_lin(edge_attr, We, be):
    E = edge_attr.shape[0]
    return pl.pallas_call(
        _edge_lin_body,
        grid=(E // EDGE_BLK,),
        in_specs=[
            pl.BlockSpec((EDGE_BLK, D_EDGE), lambda i: (i, 0)),
            pl.BlockSpec((D_EDGE, D_FEAT), lambda i: (0, 0)),
            pl.BlockSpec((1, D_FEAT), lambda i: (0, 0)),
        ],
        out_specs=pl.BlockSpec((EDGE_BLK, D_FEAT), lambda i: (i, 0)),
        out_shape=jax.ShapeDtypeStruct((E, D_FEAT), jnp.float32),
    )(edge_attr, We, be.reshape(1, D_FEAT))


def _sc_body(x_hbm, ei3_hbm, e_hbm, acc_out, cnt_out,
             idx2, dstc, xrows, erows, hist, acc_sh, sem, sem2, sem3, sem4):
    c = lax.axis_index("c")
    s = lax.axis_index("s")
    wid = s * NC + c

    zeros16 = jnp.zeros((16,), jnp.float32)

    def zero_hist(k, _):
        hist[pl.ds(k * 16, 16)] = zeros16
        return 0

    lax.fori_loop(0, (N_PAD + 16) // 16, zero_hist, 0)

    def zero_row(r, _):
        for j in range(D_FEAT // 16):
            xrows[r, pl.ds(j * 16, 16)] = zeros16
        return 0

    lax.fori_loop(0, C, zero_row, 0)

    def zero_chunk(k, _):
        pltpu.sync_copy(xrows.at[pl.ds(0, ZB)],
                        acc_sh.at[pl.ds(s * TILE_ROWS + k * ZB, ZB)])
        return 0

    lax.fori_loop(0, 9, zero_chunk, 0)
    pltpu.sync_copy(xrows.at[pl.ds(0, ZT)],
                    acc_sh.at[pl.ds(s * TILE_ROWS + 9 * ZB, ZT)])
    plsc.subcore_barrier()

    def chunk_body(i, _):
        ck = i * NW + wid

        @pl.when(ck < NCHUNK)
        def _():
            base = ck * C
            ia = pltpu.async_copy(ei3_hbm.at[ck], idx2, sem3)
            eb = pltpu.async_copy(e_hbm.at[pl.ds(base, C)], erows, sem4)

            @pl.when(i > 0)
            def _():
                pltpu.make_async_copy(x_hbm.at[pl.ds(0, C)], xrows, sem2).wait()

            ia.wait()
            gat = pltpu.async_copy(x_hbm.at[idx2.at[0]], xrows, sem)

            e0 = jnp.where(lax.iota(jnp.int32, 16) == 0, 1.0, 0.0)
            for k in range(C // 16):
                sl = pl.ds(k * 16, 16)
                dv = idx2[1, sl]
                dstc[1, sl] = dv
                for l in range(16):
                    d = dv[l]
                    hist[pl.ds(d, 16)] = hist[pl.ds(d, 16)] + e0

            eb.wait()
            gat.wait()

            def row_body(r, _):
                for j in range(D_FEAT // 16):
                    sl = pl.ds(j * 16, 16)
                    xrows[r, sl] = jnp.maximum(xrows[r, sl] + erows[r, sl], 0.0)
                return 0

            lax.fori_loop(0, C, row_body, 0)
            pltpu.async_copy(xrows, acc_sh.at[dstc.at[1]], sem2, add=True)

        return 0

    lax.fori_loop(0, CHUNK_ITERS, chunk_body, 0)
    pltpu.make_async_copy(x_hbm.at[pl.ds(0, C)], xrows, sem2).wait()
    plsc.subcore_barrier()

    def read_chunk(k, _):
        row0 = s * TILE_ROWS + k * ZB
        pltpu.sync_copy(acc_sh.at[pl.ds(row0, ZB)], xrows.at[pl.ds(0, ZB)])
        pltpu.sync_copy(xrows.at[pl.ds(0, ZB)],
                        acc_out.at[pl.ds(c * N_PAD + row0, ZB)])
        return 0

    lax.fori_loop(0, 9, read_chunk, 0)
    row9 = s * TILE_ROWS + 9 * ZB
    pltpu.sync_copy(acc_sh.at[pl.ds(row9, ZT)], xrows.at[pl.ds(0, ZT)])
    pltpu.sync_copy(xrows.at[pl.ds(0, ZT)],
                    acc_out.at[pl.ds(c * N_PAD + row9, ZT)])
    pltpu.sync_copy(hist.at[pl.ds(0, N_PAD)], cnt_out.at[wid])


def _sc_aggregate(x, ei3, e):
    mesh = plsc.VectorSubcoreMesh(
        core_axis_name="c", subcore_axis_name="s", num_cores=NC, num_subcores=NS
    )
    f = pl.kernel(
        _sc_body,
        out_type=(
            jax.ShapeDtypeStruct((NC * N_PAD, D_FEAT), jnp.float32),
            jax.ShapeDtypeStruct((NW, N_PAD), jnp.float32),
        ),
        mesh=mesh,
        scratch_types=[
            pltpu.VMEM((2, C), jnp.int32),
            pltpu.VMEM((2, C), jnp.int32),
            pltpu.VMEM((C, D_FEAT), jnp.float32),
            pltpu.VMEM((C, D_FEAT), jnp.float32),
            pltpu.VMEM((N_PAD + 16,), jnp.float32),
            pltpu.VMEM_SHARED((N_ACC, D_FEAT), jnp.float32),
            pltpu.SemaphoreType.DMA,
            pltpu.SemaphoreType.DMA,
            pltpu.SemaphoreType.DMA,
            pltpu.SemaphoreType.DMA,
        ],
    )
    return f(x, ei3, e)


def _mlp_body(x_ref, a0_ref, a1_ref, cnt_ref,
              w1_ref, b1_ref, w2_ref, b2_ref, o_ref):
    cnt = jnp.maximum(cnt_ref[...], 1.0)
    aggr = (a0_ref[...] + a1_ref[...]) / cnt
    h = x_ref[...] + aggr
    h = jnp.maximum(
        jnp.dot(h, w1_ref[...], preferred_element_type=jnp.float32) + b1_ref[...],
        0.0,
    )
    o_ref[...] = (
        jnp.dot(h, w2_ref[...], preferred_element_type=jnp.float32) + b2_ref[...]
    )


def _mlp(xp, acc, cnt, W1, b1, W2, b2):
    nb = N_PAD // ROW_BLK
    return pl.pallas_call(
        _mlp_body,
        grid=(nb,),
        in_specs=[
            pl.BlockSpec((ROW_BLK, D_FEAT), lambda i: (i, 0)),
            pl.BlockSpec((ROW_BLK, D_FEAT), lambda i: (i, 0)),
            pl.BlockSpec((ROW_BLK, D_FEAT), lambda i, nb=nb: (i + nb, 0)),
            pl.BlockSpec((ROW_BLK, 1), lambda i: (i, 0)),
            pl.BlockSpec((D_FEAT, D_FEAT), lambda i: (0, 0)),
            pl.BlockSpec((1, D_FEAT), lambda i: (0, 0)),
            pl.BlockSpec((D_FEAT, D_FEAT), lambda i: (0, 0)),
            pl.BlockSpec((1, D_FEAT), lambda i: (0, 0)),
        ],
        out_specs=pl.BlockSpec((ROW_BLK, D_FEAT), lambda i: (i, 0)),
        out_shape=jax.ShapeDtypeStruct((N_PAD, D_FEAT), jnp.float32),
    )(xp, acc, acc, cnt, W1, b1.reshape(1, D_FEAT), W2, b2.reshape(1, D_FEAT))


def kernel(x, edge_index, edge_attr, We, be, W1, b1, W2, b2):
    ei = edge_index.astype(jnp.int32)
    pad = E_PAD_SC - N_EDGES
    srcp = jnp.concatenate([ei[0], jnp.zeros((pad,), jnp.int32)])
    dstp = jnp.concatenate([ei[1], jnp.full((pad,), N_NODES, jnp.int32)])
    ei3 = jnp.stack([srcp, dstp]).reshape(2, NCHUNK, C).transpose(1, 0, 2)
    eap = jnp.pad(edge_attr, ((0, E_PAD_TC - N_EDGES), (0, 0)))
    e = _edge_lin(eap, We, be)
    acc, cnt = _sc_aggregate(x, ei3, e)
    counts = cnt.sum(axis=0).reshape(N_PAD, 1)
    xp = jnp.pad(x, ((0, N_PAD - N_NODES), (0, 0)))
    out = _mlp(xp, acc, counts, W1, b1, W2, b2)
    return out[:N_NODES]


# speedup vs baseline: 4.4792x; 1.0064x over previous
import functools

import jax
import jax.numpy as jnp
from jax import lax
from jax.experimental import pallas as pl
from jax.experimental.pallas import tpu as pltpu
from jax.experimental.pallas import tpu_sc as plsc

N_NODES = 10000
N_EDGES = 320000
D_FEAT = 128
D_EDGE = 16

EDGE_BLK = 16384

NC = 2
NS = 16
NW = NC * NS
C = 96
E_PAD_SC = ((N_EDGES + C - 1) // C) * C
NCHUNK = E_PAD_SC // C
CHUNK_ITERS = (NCHUNK + NW - 1) // NW
E_PAD_TC = ((E_PAD_SC + EDGE_BLK - 1) // EDGE_BLK) * EDGE_BLK

N_PAD = 10240
ROW_BLK = N_PAD // 4
N_ACC = 10112
TILE_ROWS = N_ACC // NS
ZB = 64
ZT = TILE_ROWS - 9 * ZB


_EPERM = []
for _m in range(D_FEAT // 32):
    for _i in range(16):
        _EPERM.append(32 * _m + _i)
        _EPERM.append(32 * _m + 16 + _i)
_EPERM = tuple(_EPERM)


def _edge_lin_body(ea_ref, we_ref, be_ref, o_ref):
    o_ref[...] = (
        jnp.dot(ea_ref[...], we_ref[...], preferred_element_type=jnp.float32)
        + be_ref[...]
    )


def _edge_lin(edge_attr, We, be):
    E = edge_attr.shape[0]
    return pl.pallas_call(
        _edge_lin_body,
        grid=(E // EDGE_BLK,),
        in_specs=[
            pl.BlockSpec((EDGE_BLK, D_EDGE), lambda i: (i, 0)),
            pl.BlockSpec((D_EDGE, D_FEAT), lambda i: (0, 0)),
            pl.BlockSpec((1, D_FEAT), lambda i: (0, 0)),
        ],
        out_specs=pl.BlockSpec((EDGE_BLK, D_FEAT), lambda i: (i, 0)),
        out_shape=jax.ShapeDtypeStruct((E, D_FEAT), jnp.float32),
    )(edge_attr, We, be.reshape(1, D_FEAT))


def _sc_body(x_hbm, ei3_hbm, e_hbm, acc_out, cnt_out,
             idx2, dstc, xrows, erows, hist, acc_sh, sem, sem2, sem3, sem4):
    c = lax.axis_index("c")
    s = lax.axis_index("s")
    wid = s * NC + c

    zeros16 = jnp.zeros((16,), jnp.float32)

    def zero_hist(k, _):
        hist[pl.ds(k * 16, 16)] = zeros16
        return 0

    lax.fori_loop(0, (N_PAD + 16) // 16, zero_hist, 0)

    def zero_row(r, _):
        for j in range(D_FEAT // 16):
            xrows[r, pl.ds(j * 16, 16)] = zeros16
        return 0

    lax.fori_loop(0, C, zero_row, 0)

    def zero_chunk(k, _):
        pltpu.sync_copy(xrows.at[pl.ds(0, ZB)],
                        acc_sh.at[pl.ds(s * TILE_ROWS + k * ZB, ZB)])
        return 0

    lax.fori_loop(0, 9, zero_chunk, 0)
    pltpu.sync_copy(xrows.at[pl.ds(0, ZT)],
                    acc_sh.at[pl.ds(s * TILE_ROWS + 9 * ZB, ZT)])
    plsc.subcore_barrier()

    def chunk_body(i, _):
        ck = i * NW + wid

        @pl.when(ck < NCHUNK)
        def _():
            base = ck * C
            ia = pltpu.async_copy(ei3_hbm.at[ck], idx2, sem3)
            eb = pltpu.async_copy(e_hbm.at[pl.ds(base, C)], erows, sem4)

            @pl.when(i > 0)
            def _():
                pltpu.make_async_copy(x_hbm.at[pl.ds(0, C)], xrows, sem2).wait()

            ia.wait()
            gat = pltpu.async_copy(x_hbm.at[idx2.at[0]], xrows, sem)

            e0 = jnp.where(lax.iota(jnp.int32, 16) == 0, 1.0, 0.0)
            for k in range(C // 16):
                sl = pl.ds(k * 16, 16)
                dv = idx2[1, sl]
                dstc[1, sl] = dv
                for l in range(16):
                    d = dv[l]
                    hist[pl.ds(d, 16)] = hist[pl.ds(d, 16)] + e0

            eb.wait()
            gat.wait()

            def row_body(r, _):
                for j in range(D_FEAT // 16):
                    sl = pl.ds(j * 16, 16)
                    xrows[r, sl] = jnp.maximum(xrows[r, sl] + erows[r, sl], 0.0)
                return 0

            lax.fori_loop(0, C, row_body, 0)
            pltpu.async_copy(xrows, acc_sh.at[dstc.at[1]], sem2, add=True)

        return 0

    lax.fori_loop(0, CHUNK_ITERS, chunk_body, 0)
    pltpu.make_async_copy(x_hbm.at[pl.ds(0, C)], xrows, sem2).wait()
    plsc.subcore_barrier()

    def read_chunk(k, _):
        row0 = s * TILE_ROWS + k * ZB
        pltpu.sync_copy(acc_sh.at[pl.ds(row0, ZB)], xrows.at[pl.ds(0, ZB)])
        pltpu.sync_copy(xrows.at[pl.ds(0, ZB)],
                        acc_out.at[pl.ds(c * N_PAD + row0, ZB)])
        return 0

    lax.fori_loop(0, 9, read_chunk, 0)
    row9 = s * TILE_ROWS + 9 * ZB
    pltpu.sync_copy(acc_sh.at[pl.ds(row9, ZT)], xrows.at[pl.ds(0, ZT)])
    pltpu.sync_copy(xrows.at[pl.ds(0, ZT)],
                    acc_out.at[pl.ds(c * N_PAD + row9, ZT)])
    pltpu.sync_copy(hist.at[pl.ds(0, N_PAD)], cnt_out.at[wid])


def _sc_aggregate(x, ei3, e):
    mesh = plsc.VectorSubcoreMesh(
        core_axis_name="c", subcore_axis_name="s", num_cores=NC, num_subcores=NS
    )
    f = pl.kernel(
        _sc_body,
        out_type=(
            jax.ShapeDtypeStruct((NC * N_PAD, D_FEAT), jnp.float32),
            jax.ShapeDtypeStruct((NW, N_PAD), jnp.float32),
        ),
        mesh=mesh,
        scratch_types=[
            pltpu.VMEM((2, C), jnp.int32),
            pltpu.VMEM((2, C), jnp.int32),
            pltpu.VMEM((C, D_FEAT), jnp.float32),
            pltpu.VMEM((C, D_FEAT), jnp.float32),
            pltpu.VMEM((N_PAD + 16,), jnp.float32),
            pltpu.VMEM_SHARED((N_ACC, D_FEAT), jnp.float32),
            pltpu.SemaphoreType.DMA,
            pltpu.SemaphoreType.DMA,
            pltpu.SemaphoreType.DMA,
            pltpu.SemaphoreType.DMA,
        ],
    )
    return f(x, ei3, e)


def _mlp_body(x_ref, a0_ref, a1_ref, cnt_ref,
              w1_ref, b1_ref, w2_ref, b2_ref, o_ref):
    cnt = jnp.maximum(cnt_ref[...], 1.0)
    aggr = (a0_ref[...] + a1_ref[...]) / cnt
    h = x_ref[...] + aggr
    h = jnp.maximum(
        jnp.dot(h, w1_ref[...], preferred_element_type=jnp.float32) + b1_ref[...],
        0.0,
    )
    o_ref[...] = (
        jnp.dot(h, w2_ref[...], preferred_element_type=jnp.float32) + b2_ref[...]
    )


def _mlp(xp, acc, cnt, W1, b1, W2, b2):
    nb = N_PAD // ROW_BLK
    return pl.pallas_call(
        _mlp_body,
        grid=(nb,),
        in_specs=[
            pl.BlockSpec((ROW_BLK, D_FEAT), lambda i: (i, 0)),
            pl.BlockSpec((ROW_BLK, D_FEAT), lambda i: (i, 0)),
            pl.BlockSpec((ROW_BLK, D_FEAT), lambda i, nb=nb: (i + nb, 0)),
            pl.BlockSpec((ROW_BLK, 1), lambda i: (i, 0)),
            pl.BlockSpec((D_FEAT, D_FEAT), lambda i: (0, 0)),
            pl.BlockSpec((1, D_FEAT), lambda i: (0, 0)),
            pl.BlockSpec((D_FEAT, D_FEAT), lambda i: (0, 0)),
            pl.BlockSpec((1, D_FEAT), lambda i: (0, 0)),
        ],
        out_specs=pl.BlockSpec((ROW_BLK, D_FEAT), lambda i: (i, 0)),
        out_shape=jax.ShapeDtypeStruct((N_PAD, D_FEAT), jnp.float32),
    )(xp, acc, acc, cnt, W1, b1.reshape(1, D_FEAT), W2, b2.reshape(1, D_FEAT))


def kernel(x, edge_index, edge_attr, We, be, W1, b1, W2, b2):
    ei = edge_index.astype(jnp.int32)
    pad = E_PAD_SC - N_EDGES
    srcp = jnp.concatenate([ei[0], jnp.zeros((pad,), jnp.int32)])
    dstp = jnp.concatenate([ei[1], jnp.full((pad,), N_NODES, jnp.int32)])
    ei3 = jnp.stack([srcp, dstp]).reshape(2, NCHUNK, C).transpose(1, 0, 2)
    eap = jnp.pad(edge_attr, ((0, E_PAD_TC - N_EDGES), (0, 0)))
    e = _edge_lin(eap, We, be)
    acc, cnt = _sc_aggregate(x, ei3, e)
    counts = cnt.sum(axis=0).reshape(N_PAD, 1)
    xp = jnp.pad(x, ((0, N_PAD - N_NODES), (0, 0)))
    out = _mlp(xp, acc, counts, W1, b1, W2, b2)
    return out[:N_NODES]
